# baseline (speedup 1.0000x reference)
import functools

import jax
import jax.numpy as jnp
from jax import lax
from jax.experimental import pallas as pl
from jax.experimental.pallas import tpu as pltpu
from jax.experimental.pallas import tpu_sc as plsc

N_ROWS = 9216
D = 64
K = 8192

BM = 1024
BK = 2048
GM = N_ROWS // BM
GK = K // BK


def _argmax_body(x_ref, cb_ref, idx_ref, bv_ref, bi_ref):
    k = pl.program_id(1)
    x = x_ref[...]
    xn = x / jnp.maximum(jnp.sqrt(jnp.sum(x * x, axis=1, keepdims=True)), 1e-12)
    cb = cb_ref[...]
    cbn = cb / jnp.maximum(jnp.sqrt(jnp.sum(cb * cb, axis=1, keepdims=True)), 1e-12)
    s = lax.dot_general(xn, cbn, (((1,), (1,)), ((), ())),
                        preferred_element_type=jnp.float32,
                        precision=lax.Precision.DEFAULT)
    tmax = jnp.max(s, axis=1, keepdims=True)
    col = lax.broadcasted_iota(jnp.int32, s.shape, 1)
    targ = jnp.min(jnp.where(s == tmax, col, K), axis=1, keepdims=True) + k * BK

    @pl.when(k == 0)
    def _():
        bv_ref[...] = tmax
        bi_ref[...] = targ

    @pl.when(k > 0)
    def _():
        better = tmax > bv_ref[...]
        bi_ref[...] = jnp.where(better, targ, bi_ref[...])
        bv_ref[...] = jnp.where(better, tmax, bv_ref[...])

    @pl.when(k == GK - 1)
    def _():
        idx_ref[...] = bi_ref[...]


def _nearest_indices(x_flat, codebook):
    return pl.pallas_call(
        _argmax_body,
        grid=(GM, GK),
        in_specs=[pl.BlockSpec((BM, D), lambda m, k: (m, 0)),
                  pl.BlockSpec((BK, D), lambda m, k: (k, 0))],
        out_specs=pl.BlockSpec((BM, 1), lambda m, k: (m, 0)),
        out_shape=jax.ShapeDtypeStruct((N_ROWS, 1), jnp.int32),
        scratch_shapes=[pltpu.VMEM((BM, 1), jnp.float32),
                        pltpu.VMEM((BM, 1), jnp.int32)],
    )(x_flat, codebook)


_NC = 2
_NS = 16
_NW = _NC * _NS
_RPW = N_ROWS // _NW
_CH = 3
_RPC = _RPW // _CH
_KPW = K // _NS
_CL = 16


def _sc_body(cb_hbm, idx_hbm, ones_hbm, zeros_hbm, zq_hbm, cnt_hbm,
             idx_v, rows_v, ones_v, cnt_v, shared, sem):
    c = lax.axis_index("c")
    s = lax.axis_index("s")
    wid = s * _NC + c
    pltpu.sync_copy(idx_hbm.at[wid], idx_v)
    pltpu.sync_copy(ones_hbm, ones_v)
    pltpu.sync_copy(zeros_hbm.at[pl.ds(s * _KPW, _KPW)],
                    shared.at[pl.ds(s * _KPW, _KPW)])
    for j in range(_CH):
        pltpu.async_copy(cb_hbm.at[idx_v.at[j]],
                         rows_v.at[pl.ds(j * _RPC, _RPC)], sem).wait()
    pltpu.sync_copy(rows_v, zq_hbm.at[pl.ds(wid * _RPW, _RPW)])
    plsc.subcore_barrier()
    for j in range(_CH):
        pltpu.sync_copy(ones_v, shared.at[idx_v.at[j]], add=True)
    plsc.subcore_barrier()
    pltpu.sync_copy(shared.at[pl.ds(s * _KPW, _KPW)], cnt_v)
    pltpu.sync_copy(cnt_v, cnt_hbm.at[c, pl.ds(s * _KPW, _KPW)])


@functools.partial(jax.jit, static_argnums=())
def _sc_gather_count(codebook, indices):
    idx3 = indices.reshape(_NW, _CH, _RPC)
    ones = jnp.ones((_RPC, _CL), jnp.float32)
    zeros = jnp.zeros((K, _CL), jnp.float32)
    run = pl.kernel(
        _sc_body,
        out_type=[jax.ShapeDtypeStruct((N_ROWS, D), jnp.float32),
                  jax.ShapeDtypeStruct((_NC, K, _CL), jnp.float32)],
        mesh=plsc.VectorSubcoreMesh(core_axis_name="c", subcore_axis_name="s"),
        scratch_types=[pltpu.VMEM((_CH, _RPC), jnp.int32),
                       pltpu.VMEM((_RPW, D), jnp.float32),
                       pltpu.VMEM((_RPC, _CL), jnp.float32),
                       pltpu.VMEM((_KPW, _CL), jnp.float32),
                       pltpu.VMEM_SHARED((K, _CL), jnp.float32),
                       pltpu.SemaphoreType.DMA],
        compiler_params=pltpu.CompilerParams(use_tc_tiling_on_sc=False),
    )
    return run(codebook, idx3, ones, zeros)


def _scalars_body(x_ref, zq_ref, cnt_ref, loss_ref, perp_ref, util_ref):
    x = x_ref[...]
    zq = zq_ref[...]
    d2 = (x - zq) ** 2
    loss_ref[...] = (jnp.sum(d2) / (N_ROWS * D))[None, None]
    c0 = cnt_ref[0, :, 0:1] + cnt_ref[1, :, 0:1]
    p = c0 / N_ROWS
    ent = jnp.sum(p * jnp.log(p + 1e-10))
    perp_ref[...] = jnp.exp(-ent)[None, None]
    util_ref[...] = (jnp.sum((c0 > 0).astype(jnp.float32)) / K)[None, None]


def _scalars(x_flat, zq_flat, counts):
    return pl.pallas_call(
        _scalars_body,
        out_shape=[jax.ShapeDtypeStruct((1, 1), jnp.float32)] * 3,
    )(x_flat, zq_flat, counts)


def kernel(x, codebook):
    shape = x.shape
    x_flat = x.reshape(-1, D)
    idx2d = _nearest_indices(x_flat, codebook)
    indices = idx2d.reshape(-1)
    zq_flat, counts = _sc_gather_count(codebook, indices)
    loss, perp, util = _scalars(x_flat, zq_flat, counts)
    return (zq_flat.reshape(shape), indices.reshape(shape[:-1]),
            loss.reshape(()), perp.reshape(()), util.reshape(()))


# speedup vs baseline: 1.2203x; 1.2203x over previous
import functools

import jax
import jax.numpy as jnp
from jax import lax
from jax.experimental import pallas as pl
from jax.experimental.pallas import tpu as pltpu
from jax.experimental.pallas import tpu_sc as plsc

N_ROWS = 9216
D = 64
K = 8192

BM = 1024
BK = 2048
GM = N_ROWS // BM
GK = K // BK


def _argmax_body(x_ref, cb_ref, col_ref, idx_ref, cbn_ref):
    m = pl.program_id(0)

    @pl.when(m == 0)
    def _():
        cb = cb_ref[...]
        cbn_ref[...] = cb / jnp.maximum(
            jnp.sqrt(jnp.sum(cb * cb, axis=1, keepdims=True)), 1e-12)

    x = x_ref[...]
    xn = x / jnp.maximum(jnp.sqrt(jnp.sum(x * x, axis=1, keepdims=True)), 1e-12)
    s = lax.dot_general(xn, cbn_ref[...], (((1,), (1,)), ((), ())),
                        preferred_element_type=jnp.float32,
                        precision=lax.Precision.DEFAULT)
    tmax = jnp.max(s, axis=1, keepdims=True)
    col = col_ref[...]
    first = jnp.min(jnp.where(s == tmax, col, jnp.float32(K)),
                    axis=1, keepdims=True)
    idx_ref[...] = first.astype(jnp.int32)


def _nearest_indices(x_flat, codebook):
    col_row = jnp.arange(K, dtype=jnp.float32)[None, :]
    return pl.pallas_call(
        _argmax_body,
        grid=(GM,),
        in_specs=[pl.BlockSpec((BM, D), lambda m: (m, 0)),
                  pl.BlockSpec((K, D), lambda m: (0, 0)),
                  pl.BlockSpec((1, K), lambda m: (0, 0))],
        out_specs=pl.BlockSpec((BM, 1), lambda m: (m, 0)),
        out_shape=jax.ShapeDtypeStruct((N_ROWS, 1), jnp.int32),
        scratch_shapes=[pltpu.VMEM((K, D), jnp.float32)],
    )(x_flat, codebook, col_row)


_NC = 2
_NS = 16
_NW = _NC * _NS
_RPW = N_ROWS // _NW
_CH = 3
_RPC = _RPW // _CH
_KPW = K // _NS
_CL = 16


def _sc_body(cb_hbm, idx_hbm, ones_hbm, zeros_hbm, zq_hbm, cnt_hbm,
             idx_v, rows_v, ones_v, cnt_v, shared, sem):
    c = lax.axis_index("c")
    s = lax.axis_index("s")
    wid = s * _NC + c
    pltpu.sync_copy(idx_hbm.at[wid], idx_v)
    pltpu.sync_copy(ones_hbm, ones_v)
    pltpu.sync_copy(zeros_hbm.at[pl.ds(s * _KPW, _KPW)],
                    shared.at[pl.ds(s * _KPW, _KPW)])
    for j in range(_CH):
        pltpu.async_copy(cb_hbm.at[idx_v.at[j]],
                         rows_v.at[pl.ds(j * _RPC, _RPC)], sem).wait()
    pltpu.sync_copy(rows_v, zq_hbm.at[pl.ds(wid * _RPW, _RPW)])
    plsc.subcore_barrier()
    for j in range(_CH):
        pltpu.sync_copy(ones_v, shared.at[idx_v.at[j]], add=True)
    plsc.subcore_barrier()
    pltpu.sync_copy(shared.at[pl.ds(s * _KPW, _KPW)], cnt_v)
    pltpu.sync_copy(cnt_v, cnt_hbm.at[c, pl.ds(s * _KPW, _KPW)])


@functools.partial(jax.jit, static_argnums=())
def _sc_gather_count(codebook, indices):
    idx3 = indices.reshape(_NW, _CH, _RPC)
    ones = jnp.ones((_RPC, _CL), jnp.float32)
    zeros = jnp.zeros((K, _CL), jnp.float32)
    run = pl.kernel(
        _sc_body,
        out_type=[jax.ShapeDtypeStruct((N_ROWS, D), jnp.float32),
                  jax.ShapeDtypeStruct((_NC, K, _CL), jnp.float32)],
        mesh=plsc.VectorSubcoreMesh(core_axis_name="c", subcore_axis_name="s"),
        scratch_types=[pltpu.VMEM((_CH, _RPC), jnp.int32),
                       pltpu.VMEM((_RPW, D), jnp.float32),
                       pltpu.VMEM((_RPC, _CL), jnp.float32),
                       pltpu.VMEM((_KPW, _CL), jnp.float32),
                       pltpu.VMEM_SHARED((K, _CL), jnp.float32),
                       pltpu.SemaphoreType.DMA],
        compiler_params=pltpu.CompilerParams(use_tc_tiling_on_sc=False),
    )
    return run(codebook, idx3, ones, zeros)


def _scalars_body(x_ref, zq_ref, cnt_ref, loss_ref, perp_ref, util_ref):
    x = x_ref[...]
    zq = zq_ref[...]
    d2 = (x - zq) ** 2
    loss_ref[...] = (jnp.sum(d2) / (N_ROWS * D))[None, None]
    c0 = cnt_ref[0, :, 0:1] + cnt_ref[1, :, 0:1]
    p = c0 / N_ROWS
    ent = jnp.sum(p * jnp.log(p + 1e-10))
    perp_ref[...] = jnp.exp(-ent)[None, None]
    util_ref[...] = (jnp.sum((c0 > 0).astype(jnp.float32)) / K)[None, None]


def _scalars(x_flat, zq_flat, counts):
    return pl.pallas_call(
        _scalars_body,
        out_shape=[jax.ShapeDtypeStruct((1, 1), jnp.float32)] * 3,
    )(x_flat, zq_flat, counts)


def kernel(x, codebook):
    shape = x.shape
    x_flat = x.reshape(-1, D)
    idx2d = _nearest_indices(x_flat, codebook)
    indices = idx2d.reshape(-1)
    zq_flat, counts = _sc_gather_count(codebook, indices)
    loss, perp, util = _scalars(x_flat, zq_flat, counts)
    return (zq_flat.reshape(shape), indices.reshape(shape[:-1]),
            loss.reshape(()), perp.reshape(()), util.reshape(()))


# speedup vs baseline: 1.5081x; 1.2358x over previous
import functools

import jax
import jax.numpy as jnp
from jax import lax
from jax.experimental import pallas as pl
from jax.experimental.pallas import tpu as pltpu
from jax.experimental.pallas import tpu_sc as plsc

N_ROWS = 9216
D = 64
K = 8192

BM = 1024
BK = 2048
GM = N_ROWS // BM
GK = K // BK


_NBK = 2048
_SB = 128


def _argmax_body(x_ref, cb_ref, idx_ref, cbn_ref):
    m = pl.program_id(0)

    @pl.when(m == 0)
    def _():
        cb = cb_ref[...]
        cbn_ref[...] = cb / jnp.maximum(
            jnp.sqrt(jnp.sum(cb * cb, axis=1, keepdims=True)), 1e-12)

    x = x_ref[...]
    xn = x / jnp.maximum(jnp.sqrt(jnp.sum(x * x, axis=1, keepdims=True)), 1e-12)
    best_v = jnp.full((BM, _SB), -jnp.inf, jnp.float32)
    best_b = jnp.zeros((BM, _SB), jnp.float32)
    for c in range(K // _NBK):
        cbn_c = cbn_ref[pl.ds(c * _NBK, _NBK), :]
        s = lax.dot_general(xn, cbn_c, (((1,), (1,)), ((), ())),
                            preferred_element_type=jnp.float32,
                            precision=lax.Precision.DEFAULT)
        for j in range(_NBK // _SB):
            blk = s[:, j * _SB:(j + 1) * _SB]
            b_id = jnp.float32(c * (_NBK // _SB) + j)
            gt = blk > best_v
            best_v = jnp.maximum(blk, best_v)
            best_b = jnp.where(gt, b_id, best_b)
    tmax = jnp.max(best_v, axis=1, keepdims=True)
    lane = lax.broadcasted_iota(jnp.int32, (BM, _SB), 1).astype(jnp.float32)
    colf = best_b * _SB + lane
    first = jnp.min(jnp.where(best_v == tmax, colf, jnp.float32(K)),
                    axis=1, keepdims=True)
    idx_ref[...] = first.astype(jnp.int32)


def _nearest_indices(x_flat, codebook):
    return pl.pallas_call(
        _argmax_body,
        grid=(GM,),
        in_specs=[pl.BlockSpec((BM, D), lambda m: (m, 0)),
                  pl.BlockSpec((K, D), lambda m: (0, 0))],
        out_specs=pl.BlockSpec((BM, 1), lambda m: (m, 0)),
        out_shape=jax.ShapeDtypeStruct((N_ROWS, 1), jnp.int32),
        scratch_shapes=[pltpu.VMEM((K, D), jnp.float32)],
    )(x_flat, codebook)


_NC = 2
_NS = 16
_NW = _NC * _NS
_RPW = N_ROWS // _NW
_CH = 3
_RPC = _RPW // _CH
_KPW = K // _NS
_CL = 16


def _sc_body(cb_hbm, idx_hbm, ones_hbm, zeros_hbm, zq_hbm, cnt_hbm,
             idx_v, rows_v, ones_v, cnt_v, shared, sem):
    c = lax.axis_index("c")
    s = lax.axis_index("s")
    wid = s * _NC + c
    pltpu.sync_copy(idx_hbm.at[wid], idx_v)
    pltpu.sync_copy(ones_hbm, ones_v)
    pltpu.sync_copy(zeros_hbm.at[pl.ds(s * _KPW, _KPW)],
                    shared.at[pl.ds(s * _KPW, _KPW)])
    for j in range(_CH):
        pltpu.async_copy(cb_hbm.at[idx_v.at[j]],
                         rows_v.at[pl.ds(j * _RPC, _RPC)], sem).wait()
    pltpu.sync_copy(rows_v, zq_hbm.at[pl.ds(wid * _RPW, _RPW)])
    plsc.subcore_barrier()
    for j in range(_CH):
        pltpu.sync_copy(ones_v, shared.at[idx_v.at[j]], add=True)
    plsc.subcore_barrier()
    pltpu.sync_copy(shared.at[pl.ds(s * _KPW, _KPW)], cnt_v)
    pltpu.sync_copy(cnt_v, cnt_hbm.at[c, pl.ds(s * _KPW, _KPW)])


@functools.partial(jax.jit, static_argnums=())
def _sc_gather_count(codebook, indices):
    idx3 = indices.reshape(_NW, _CH, _RPC)
    ones = jnp.ones((_RPC, _CL), jnp.float32)
    zeros = jnp.zeros((K, _CL), jnp.float32)
    run = pl.kernel(
        _sc_body,
        out_type=[jax.ShapeDtypeStruct((N_ROWS, D), jnp.float32),
                  jax.ShapeDtypeStruct((_NC, K, _CL), jnp.float32)],
        mesh=plsc.VectorSubcoreMesh(core_axis_name="c", subcore_axis_name="s"),
        scratch_types=[pltpu.VMEM((_CH, _RPC), jnp.int32),
                       pltpu.VMEM((_RPW, D), jnp.float32),
                       pltpu.VMEM((_RPC, _CL), jnp.float32),
                       pltpu.VMEM((_KPW, _CL), jnp.float32),
                       pltpu.VMEM_SHARED((K, _CL), jnp.float32),
                       pltpu.SemaphoreType.DMA],
        compiler_params=pltpu.CompilerParams(use_tc_tiling_on_sc=False),
    )
    return run(codebook, idx3, ones, zeros)


def _scalars_body(x_ref, zq_ref, cnt_ref, loss_ref, perp_ref, util_ref):
    x = x_ref[...]
    zq = zq_ref[...]
    d2 = (x - zq) ** 2
    loss_ref[...] = (jnp.sum(d2) / (N_ROWS * D))[None, None]
    c0 = cnt_ref[0, :, 0:1] + cnt_ref[1, :, 0:1]
    p = c0 / N_ROWS
    ent = jnp.sum(p * jnp.log(p + 1e-10))
    perp_ref[...] = jnp.exp(-ent)[None, None]
    util_ref[...] = (jnp.sum((c0 > 0).astype(jnp.float32)) / K)[None, None]


def _scalars(x_flat, zq_flat, counts):
    return pl.pallas_call(
        _scalars_body,
        out_shape=[jax.ShapeDtypeStruct((1, 1), jnp.float32)] * 3,
    )(x_flat, zq_flat, counts)


def kernel(x, codebook):
    shape = x.shape
    x_flat = x.reshape(-1, D)
    idx2d = _nearest_indices(x_flat, codebook)
    indices = idx2d.reshape(-1)
    zq_flat, counts = _sc_gather_count(codebook, indices)
    loss, perp, util = _scalars(x_flat, zq_flat, counts)
    return (zq_flat.reshape(shape), indices.reshape(shape[:-1]),
            loss.reshape(()), perp.reshape(()), util.reshape(()))


# speedup vs baseline: 1.7191x; 1.1399x over previous
import numpy as np

import jax
import jax.numpy as jnp
from jax import lax
from jax.experimental import pallas as pl
from jax.experimental.pallas import tpu as pltpu
from jax.experimental.pallas import tpu_sc as plsc

N_ROWS = 9216
D = 64
K = 8192

BM = 1024
GM = N_ROWS // BM
_NBK = 2048
_SB = 128


def _argmax_body(x_ref, cb_ref, idx_ref, cbn_ref):
    m = pl.program_id(0)

    @pl.when(m == 0)
    def _():
        cb = cb_ref[...]
        cbn_ref[...] = cb / jnp.maximum(
            jnp.sqrt(jnp.sum(cb * cb, axis=1, keepdims=True)), 1e-12)

    x = x_ref[...]
    xn = x / jnp.maximum(jnp.sqrt(jnp.sum(x * x, axis=1, keepdims=True)), 1e-12)
    best_v = jnp.full((BM, _SB), -jnp.inf, jnp.float32)
    best_b = jnp.zeros((BM, _SB), jnp.float32)
    for c in range(K // _NBK):
        cbn_c = cbn_ref[pl.ds(c * _NBK, _NBK), :]
        s = lax.dot_general(xn, cbn_c, (((1,), (1,)), ((), ())),
                            preferred_element_type=jnp.float32,
                            precision=lax.Precision.DEFAULT)
        for j in range(_NBK // _SB):
            blk = s[:, j * _SB:(j + 1) * _SB]
            b_id = jnp.float32(c * (_NBK // _SB) + j)
            gt = blk > best_v
            best_v = jnp.maximum(blk, best_v)
            best_b = jnp.where(gt, b_id, best_b)
    tmax = jnp.max(best_v, axis=1, keepdims=True)
    lane = lax.broadcasted_iota(jnp.int32, (BM, _SB), 1).astype(jnp.float32)
    colf = best_b * _SB + lane
    first = jnp.min(jnp.where(best_v == tmax, colf, jnp.float32(K)),
                    axis=1, keepdims=True)
    idx_ref[...] = first.astype(jnp.int32).reshape(BM // 128, 128)


def _nearest_indices(x_flat, codebook):
    return pl.pallas_call(
        _argmax_body,
        grid=(GM,),
        in_specs=[pl.BlockSpec((BM, D), lambda m: (m, 0)),
                  pl.BlockSpec((K, D), lambda m: (0, 0))],
        out_specs=pl.BlockSpec((BM // 128, 128), lambda m: (m, 0)),
        out_shape=jax.ShapeDtypeStruct((N_ROWS // 128, 128), jnp.int32),
        scratch_shapes=[pltpu.VMEM((K, D), jnp.float32)],
    )(x_flat, codebook)


_NC = 2
_NS = 16
_NW = _NC * _NS
_RPW = N_ROWS // _NW
_CH = 3
_RPC = _RPW // _CH
_KPW = K // _NS
_CL = 16

_ONES = np.ones((_RPC, _CL), np.float32)
_ZEROS = np.zeros((K, _CL), np.float32)


def _sc_body(cb_hbm, idx_hbm, ones_hbm, zeros_hbm, zq_hbm, cnt_hbm,
             idx_v, rows_v, ones_v, cnt_v, shared, sem):
    c = lax.axis_index("c")
    s = lax.axis_index("s")
    wid = s * _NC + c
    pltpu.sync_copy(idx_hbm.at[wid], idx_v)
    pltpu.sync_copy(ones_hbm, ones_v)
    pltpu.sync_copy(zeros_hbm.at[pl.ds(s * _KPW, _KPW)],
                    shared.at[pl.ds(s * _KPW, _KPW)])
    for j in range(_CH):
        pltpu.async_copy(cb_hbm.at[idx_v.at[j]],
                         rows_v.at[pl.ds(j * _RPC, _RPC)], sem).wait()
    pltpu.sync_copy(rows_v, zq_hbm.at[pl.ds(wid * _RPW, _RPW)])
    plsc.subcore_barrier()
    for j in range(_CH):
        pltpu.sync_copy(ones_v, shared.at[idx_v.at[j]], add=True)
    plsc.subcore_barrier()
    pltpu.sync_copy(shared.at[pl.ds(s * _KPW, _KPW)], cnt_v)
    pltpu.sync_copy(cnt_v, cnt_hbm.at[c, pl.ds(s * _KPW, _KPW)])


def _sc_gather_count(codebook, idx3):
    run = pl.kernel(
        _sc_body,
        out_type=[jax.ShapeDtypeStruct((N_ROWS, D), jnp.float32),
                  jax.ShapeDtypeStruct((_NC, K, _CL), jnp.float32)],
        mesh=plsc.VectorSubcoreMesh(core_axis_name="c", subcore_axis_name="s"),
        scratch_types=[pltpu.VMEM((_CH, _RPC), jnp.int32),
                       pltpu.VMEM((_RPW, D), jnp.float32),
                       pltpu.VMEM((_RPC, _CL), jnp.float32),
                       pltpu.VMEM((_KPW, _CL), jnp.float32),
                       pltpu.VMEM_SHARED((K, _CL), jnp.float32),
                       pltpu.SemaphoreType.DMA],
        compiler_params=pltpu.CompilerParams(use_tc_tiling_on_sc=False),
    )
    return run(codebook, idx3, _ONES, _ZEROS)


def _scalars_body(x_ref, zq_ref, cnt_ref, zq_out_ref, loss_ref, perp_ref,
                  util_ref):
    x = x_ref[...]
    zq = zq_ref[...]
    zq_out_ref[...] = zq.reshape(16, N_ROWS // 16, D)
    d2 = (x - zq) ** 2
    loss_ref[...] = (jnp.sum(d2) / (N_ROWS * D))[None, None]
    cnt = cnt_ref[0:K // 8, :] + cnt_ref[K // 8:, :]
    p = cnt / N_ROWS
    ent = jnp.sum(p * jnp.log(p + 1e-10)) / _CL
    perp_ref[...] = jnp.exp(-ent)[None, None]
    util_ref[...] = (jnp.sum((cnt > 0).astype(jnp.float32)) / (_CL * K))[None, None]


def _scalars(x_flat, zq_flat, cnt2):
    return pl.pallas_call(
        _scalars_body,
        out_shape=[jax.ShapeDtypeStruct((16, N_ROWS // 16, D), jnp.float32),
                   jax.ShapeDtypeStruct((1, 1), jnp.float32),
                   jax.ShapeDtypeStruct((1, 1), jnp.float32),
                   jax.ShapeDtypeStruct((1, 1), jnp.float32)],
    )(x_flat, zq_flat, cnt2)


def kernel(x, codebook):
    shape = x.shape
    x_flat = x.reshape(-1, D)
    idx2d = _nearest_indices(x_flat, codebook)
    idx3 = idx2d.reshape(_NW, _CH, _RPC)
    zq_flat, counts = _sc_gather_count(codebook, idx3)
    cnt2 = counts.reshape(_NC * K * _CL // 128, 128)
    zq_out, loss, perp, util = _scalars(x_flat, zq_flat, cnt2)
    return (zq_out.reshape(shape), idx2d.reshape(shape[:-1]),
            loss.reshape(()), perp.reshape(()), util.reshape(()))


# speedup vs baseline: 1.7429x; 1.0138x over previous
import numpy as np

import jax
import jax.numpy as jnp
from jax import lax
from jax.experimental import pallas as pl
from jax.experimental.pallas import tpu as pltpu
from jax.experimental.pallas import tpu_sc as plsc

N_ROWS = 9216
D = 64
K = 8192

BM = 1024
GM = N_ROWS // BM
_NBK = 2048
_SB = 128


def _argmax_body(x_ref, cb_ref, idx_ref, cbn_ref):
    m = pl.program_id(0)

    @pl.when(m == 0)
    def _():
        cb = cb_ref[...]
        cbn_ref[...] = cb / jnp.maximum(
            jnp.sqrt(jnp.sum(cb * cb, axis=1, keepdims=True)), 1e-12)

    x = x_ref[...]
    xn = x / jnp.maximum(jnp.sqrt(jnp.sum(x * x, axis=1, keepdims=True)), 1e-12)
    best_v = jnp.full((BM, _SB), -jnp.inf, jnp.float32)
    best_b = jnp.zeros((BM, _SB), jnp.float32)
    for c in range(K // _NBK):
        cbn_c = cbn_ref[pl.ds(c * _NBK, _NBK), :]
        s = lax.dot_general(xn, cbn_c, (((1,), (1,)), ((), ())),
                            preferred_element_type=jnp.float32,
                            precision=lax.Precision.DEFAULT)
        for j in range(_NBK // _SB):
            blk = s[:, j * _SB:(j + 1) * _SB]
            b_id = jnp.float32(c * (_NBK // _SB) + j)
            gt = blk > best_v
            best_v = jnp.maximum(blk, best_v)
            best_b = jnp.where(gt, b_id, best_b)
    tmax = jnp.max(best_v, axis=1, keepdims=True)
    lane = lax.broadcasted_iota(jnp.int32, (BM, _SB), 1).astype(jnp.float32)
    colf = best_b * _SB + lane
    first = jnp.min(jnp.where(best_v == tmax, colf, jnp.float32(K)),
                    axis=1, keepdims=True)
    idx_ref[...] = first.astype(jnp.int32).reshape(BM // 128, 128)


def _nearest_indices(x_flat, codebook):
    return pl.pallas_call(
        _argmax_body,
        grid=(GM,),
        in_specs=[pl.BlockSpec((BM, D), lambda m: (m, 0)),
                  pl.BlockSpec((K, D), lambda m: (0, 0))],
        out_specs=pl.BlockSpec((BM // 128, 128), lambda m: (m, 0)),
        out_shape=jax.ShapeDtypeStruct((N_ROWS // 128, 128), jnp.int32),
        scratch_shapes=[pltpu.VMEM((K, D), jnp.float32)],
    )(x_flat, codebook)


_NC = 2
_NS = 16
_NW = _NC * _NS
_RPW = N_ROWS // _NW
_CH = 3
_RPC = _RPW // _CH
_KPW = K // _NS
_CL = 16

def _sc_body(cb_hbm, idx_hbm, zq_hbm, cnt_hbm,
             idx_v, rows_v, ones_v, cnt_v, shared, sem):
    c = lax.axis_index("c")
    s = lax.axis_index("s")
    wid = s * _NC + c
    pltpu.sync_copy(idx_hbm.at[wid], idx_v)

    def _fill(i, _):
        ones_v[i, :] = jnp.full((_CL,), 1.0, jnp.float32)
        return 0

    def _zero(i, _):
        cnt_v[i, :] = jnp.zeros((_CL,), jnp.float32)
        return 0

    lax.fori_loop(0, _RPC, _fill, 0)
    lax.fori_loop(0, _KPW, _zero, 0)
    pltpu.sync_copy(cnt_v, shared.at[pl.ds(s * _KPW, _KPW)])
    for j in range(_CH):
        pltpu.async_copy(cb_hbm.at[idx_v.at[j]],
                         rows_v.at[pl.ds(j * _RPC, _RPC)], sem).wait()
    pltpu.sync_copy(rows_v, zq_hbm.at[pl.ds(wid * _RPW, _RPW)])
    plsc.subcore_barrier()
    for j in range(_CH):
        pltpu.sync_copy(ones_v, shared.at[idx_v.at[j]], add=True)
    plsc.subcore_barrier()
    pltpu.sync_copy(shared.at[pl.ds(s * _KPW, _KPW)], cnt_v)
    pltpu.sync_copy(cnt_v, cnt_hbm.at[c, pl.ds(s * _KPW, _KPW)])


def _sc_gather_count(codebook, idx3):
    run = pl.kernel(
        _sc_body,
        out_type=[jax.ShapeDtypeStruct((N_ROWS, D), jnp.float32),
                  jax.ShapeDtypeStruct((_NC, K, _CL), jnp.float32)],
        mesh=plsc.VectorSubcoreMesh(core_axis_name="c", subcore_axis_name="s"),
        scratch_types=[pltpu.VMEM((_CH, _RPC), jnp.int32),
                       pltpu.VMEM((_RPW, D), jnp.float32),
                       pltpu.VMEM((_RPC, _CL), jnp.float32),
                       pltpu.VMEM((_KPW, _CL), jnp.float32),
                       pltpu.VMEM_SHARED((K, _CL), jnp.float32),
                       pltpu.SemaphoreType.DMA],
        compiler_params=pltpu.CompilerParams(use_tc_tiling_on_sc=False),
    )
    return run(codebook, idx3)


def _scalars_body(x_ref, zq_ref, cnt_ref, zq_out_ref, loss_ref, perp_ref,
                  util_ref):
    x = x_ref[...]
    zq = zq_ref[...]
    zq_out_ref[...] = zq.reshape(16, N_ROWS // 16, D)
    d2 = (x - zq) ** 2
    loss_ref[...] = (jnp.sum(d2) / (N_ROWS * D))[None, None]
    cnt = cnt_ref[0:K // 8, :] + cnt_ref[K // 8:, :]
    p = cnt / N_ROWS
    ent = jnp.sum(p * jnp.log(p + 1e-10)) / _CL
    perp_ref[...] = jnp.exp(-ent)[None, None]
    util_ref[...] = (jnp.sum((cnt > 0).astype(jnp.float32)) / (_CL * K))[None, None]


def _scalars(x_flat, zq_flat, cnt2):
    return pl.pallas_call(
        _scalars_body,
        out_shape=[jax.ShapeDtypeStruct((16, N_ROWS // 16, D), jnp.float32),
                   jax.ShapeDtypeStruct((1, 1), jnp.float32),
                   jax.ShapeDtypeStruct((1, 1), jnp.float32),
                   jax.ShapeDtypeStruct((1, 1), jnp.float32)],
    )(x_flat, zq_flat, cnt2)


def kernel(x, codebook):
    shape = x.shape
    x_flat = x.reshape(-1, D)
    idx2d = _nearest_indices(x_flat, codebook)
    idx3 = idx2d.reshape(_NW, _CH, _RPC)
    zq_flat, counts = _sc_gather_count(codebook, idx3)
    cnt2 = counts.reshape(_NC * K * _CL // 128, 128)
    zq_out, loss, perp, util = _scalars(x_flat, zq_flat, cnt2)
    del shape
    return (zq_out, idx2d.reshape(x.shape[:-1]),
            loss.reshape(()), perp.reshape(()), util.reshape(()))


# speedup vs baseline: 1.8426x; 1.0573x over previous
import numpy as np

import jax
import jax.numpy as jnp
from jax import lax
from jax.experimental import pallas as pl
from jax.experimental.pallas import tpu as pltpu
from jax.experimental.pallas import tpu_sc as plsc

N_ROWS = 9216
D = 64
K = 8192

BM = 1024
GM = N_ROWS // BM
_NBK = 2048
_SB = 128


def _argmax_body(x_ref, cb_ref, idx_ref, cbn_ref):
    m = pl.program_id(0)

    @pl.when(m == 0)
    def _():
        cb = cb_ref[...]
        cbn_ref[...] = cb / jnp.maximum(
            jnp.sqrt(jnp.sum(cb * cb, axis=1, keepdims=True)), 1e-12)

    x = x_ref[...]
    xn = x / jnp.maximum(jnp.sqrt(jnp.sum(x * x, axis=1, keepdims=True)), 1e-12)
    best_v = jnp.full((BM, _SB), -jnp.inf, jnp.float32)
    best_b = jnp.zeros((BM, _SB), jnp.float32)
    for c in range(K // _NBK):
        cbn_c = cbn_ref[pl.ds(c * _NBK, _NBK), :]
        s = lax.dot_general(xn, cbn_c, (((1,), (1,)), ((), ())),
                            preferred_element_type=jnp.float32,
                            precision=lax.Precision.DEFAULT)
        for j in range(_NBK // _SB):
            blk = s[:, j * _SB:(j + 1) * _SB]
            b_id = jnp.float32(c * (_NBK // _SB) + j)
            gt = blk > best_v
            best_v = jnp.maximum(blk, best_v)
            best_b = jnp.where(gt, b_id, best_b)
    tmax = jnp.max(best_v, axis=1, keepdims=True)
    lane = lax.broadcasted_iota(jnp.int32, (BM, _SB), 1).astype(jnp.float32)
    colf = best_b * _SB + lane
    first = jnp.min(jnp.where(best_v == tmax, colf, jnp.float32(K)),
                    axis=1, keepdims=True)
    idx_ref[...] = first.astype(jnp.int32).reshape(BM // 128, 128)


def _nearest_indices(x_flat, codebook):
    return pl.pallas_call(
        _argmax_body,
        grid=(GM,),
        in_specs=[pl.BlockSpec((BM, D), lambda m: (m, 0)),
                  pl.BlockSpec((K, D), lambda m: (0, 0))],
        out_specs=pl.BlockSpec((BM // 128, 128), lambda m: (m, 0)),
        out_shape=jax.ShapeDtypeStruct((N_ROWS // 128, 128), jnp.int32),
        scratch_shapes=[pltpu.VMEM((K, D), jnp.float32)],
    )(x_flat, codebook)


_NC = 2
_NS = 16
_NW = _NC * _NS
_RPW = N_ROWS // _NW
_CH = 3
_RPC = _RPW // _CH
_KPW = K // _NS
_CL = 16

def _sc_body(cb_hbm, idx_hbm, zq_hbm, cnt_hbm,
             idx_v, rows_v, ones_v, cnt_v, shared, sem):
    c = lax.axis_index("c")
    s = lax.axis_index("s")
    wid = s * _NC + c
    pltpu.sync_copy(idx_hbm.at[wid], idx_v)

    def _fill(i, _):
        ones_v[i, :] = jnp.full((_CL,), 1.0, jnp.float32)
        return 0

    def _zero(i, _):
        cnt_v[i, :] = jnp.zeros((_CL,), jnp.float32)
        return 0

    lax.fori_loop(0, _RPC, _fill, 0)
    lax.fori_loop(0, _KPW, _zero, 0)
    pltpu.sync_copy(cnt_v, shared.at[pl.ds(s * _KPW, _KPW)])
    for j in range(_CH):
        pltpu.async_copy(cb_hbm.at[idx_v.at[j]],
                         rows_v.at[pl.ds(j * _RPC, _RPC)], sem).wait()
    pltpu.sync_copy(rows_v, zq_hbm.at[pl.ds(wid * _RPW, _RPW)])
    plsc.subcore_barrier()
    for j in range(_CH):
        pltpu.sync_copy(ones_v, shared.at[idx_v.at[j]], add=True)
    plsc.subcore_barrier()
    pltpu.sync_copy(shared.at[pl.ds(s * _KPW, _KPW)], cnt_v)
    pltpu.sync_copy(cnt_v, cnt_hbm.at[c, pl.ds(s * _KPW, _KPW)])


def _sc_gather_count(codebook, idx3):
    run = pl.kernel(
        _sc_body,
        out_type=[jax.ShapeDtypeStruct((N_ROWS, D), jnp.float32),
                  jax.ShapeDtypeStruct((_NC, K, _CL), jnp.float32)],
        mesh=plsc.VectorSubcoreMesh(core_axis_name="c", subcore_axis_name="s"),
        scratch_types=[pltpu.VMEM((_CH, _RPC), jnp.int32),
                       pltpu.VMEM((_RPW, D), jnp.float32),
                       pltpu.VMEM((_RPC, _CL), jnp.float32),
                       pltpu.VMEM((_KPW, _CL), jnp.float32),
                       pltpu.VMEM_SHARED((K, _CL), jnp.float32),
                       pltpu.SemaphoreType.DMA],
        compiler_params=pltpu.CompilerParams(use_tc_tiling_on_sc=False),
    )
    return run(codebook, idx3)


def _scalars_body(x_ref, zq_ref, cnt_ref, zq_out_ref, loss_ref, perp_ref,
                  util_ref):
    x = x_ref[...]
    zq = zq_ref[...]
    zq_out_ref[...] = jnp.transpose(zq.reshape(16, N_ROWS // 16, D), (0, 2, 1))
    d2 = (x - zq) ** 2
    loss_ref[...] = (jnp.sum(d2) / (N_ROWS * D))[None, None]
    cnt = cnt_ref[0:K // 8, :] + cnt_ref[K // 8:, :]
    p = cnt / N_ROWS
    ent = jnp.sum(p * jnp.log(p + 1e-10)) / _CL
    perp_ref[...] = jnp.exp(-ent)[None, None]
    util_ref[...] = (jnp.sum((cnt > 0).astype(jnp.float32)) / (_CL * K))[None, None]


def _scalars(x_flat, zq_flat, cnt2):
    return pl.pallas_call(
        _scalars_body,
        out_shape=[jax.ShapeDtypeStruct((16, D, N_ROWS // 16), jnp.float32),
                   jax.ShapeDtypeStruct((1, 1), jnp.float32),
                   jax.ShapeDtypeStruct((1, 1), jnp.float32),
                   jax.ShapeDtypeStruct((1, 1), jnp.float32)],
    )(x_flat, zq_flat, cnt2)


def kernel(x, codebook):
    shape = x.shape
    x_flat = x.reshape(-1, D)
    idx2d = _nearest_indices(x_flat, codebook)
    idx3 = idx2d.reshape(_NW, _CH, _RPC)
    zq_flat, counts = _sc_gather_count(codebook, idx3)
    cnt2 = counts.reshape(_NC * K * _CL // 128, 128)
    zq_t, loss, perp, util = _scalars(x_flat, zq_flat, cnt2)
    zq_out = jnp.transpose(zq_t, (0, 2, 1))
    del shape
    return (zq_out, idx2d.reshape(x.shape[:-1]),
            loss.reshape(()), perp.reshape(()), util.reshape(()))


# speedup vs baseline: 1.8902x; 1.0258x over previous
import numpy as np

import jax
import jax.numpy as jnp
from jax import lax
from jax.experimental import pallas as pl
from jax.experimental.pallas import tpu as pltpu
from jax.experimental.pallas import tpu_sc as plsc

N_ROWS = 9216
D = 64
K = 8192

BM = 1024
GM = N_ROWS // BM
_NBK = 2048
_SB = 128


def _argmax_body(x_ref, cbt_ref, idx_ref, cbn_ref):
    m = pl.program_id(0)

    @pl.when(m == 0)
    def _():
        cbt = cbt_ref[...]
        cbn_ref[...] = cbt / jnp.maximum(
            jnp.sqrt(jnp.sum(cbt * cbt, axis=0, keepdims=True)), 1e-12)

    x = x_ref[...]
    xn = x / jnp.maximum(jnp.sqrt(jnp.sum(x * x, axis=1, keepdims=True)), 1e-12)
    best_v = jnp.full((BM, _SB), -jnp.inf, jnp.float32)
    best_b = jnp.zeros((BM, _SB), jnp.float32)
    for c in range(K // _NBK):
        cbn_c = cbn_ref[:, pl.ds(c * _NBK, _NBK)]
        s = lax.dot_general(xn, cbn_c, (((1,), (0,)), ((), ())),
                            preferred_element_type=jnp.float32,
                            precision=lax.Precision.DEFAULT)
        for j in range(_NBK // _SB):
            blk = s[:, j * _SB:(j + 1) * _SB]
            b_id = jnp.float32(c * (_NBK // _SB) + j)
            gt = blk > best_v
            best_v = jnp.maximum(blk, best_v)
            best_b = jnp.where(gt, b_id, best_b)
    tmax = jnp.max(best_v, axis=1, keepdims=True)
    lane = lax.broadcasted_iota(jnp.int32, (BM, _SB), 1).astype(jnp.float32)
    colf = best_b * _SB + lane
    first = jnp.min(jnp.where(best_v == tmax, colf, jnp.float32(K)),
                    axis=1, keepdims=True)
    idx_ref[...] = first.astype(jnp.int32).reshape(BM // 128, 128)


def _nearest_indices(x_flat, codebook_t):
    return pl.pallas_call(
        _argmax_body,
        grid=(GM,),
        in_specs=[pl.BlockSpec((BM, D), lambda m: (m, 0)),
                  pl.BlockSpec((D, K), lambda m: (0, 0))],
        out_specs=pl.BlockSpec((BM // 128, 128), lambda m: (m, 0)),
        out_shape=jax.ShapeDtypeStruct((N_ROWS // 128, 128), jnp.int32),
        scratch_shapes=[pltpu.VMEM((D, K), jnp.float32)],
    )(x_flat, codebook_t)


_NC = 2
_NS = 16
_NW = _NC * _NS
_RPW = N_ROWS // _NW
_CH = 3
_RPC = _RPW // _CH
_KPW = K // _NS
_CL = 16

def _sc_body(cb_hbm, idx_hbm, zq_hbm, cnt_hbm,
             idx_v, rows_v, ones_v, cnt_v, shared, sem):
    c = lax.axis_index("c")
    s = lax.axis_index("s")
    wid = s * _NC + c
    pltpu.sync_copy(idx_hbm.at[wid], idx_v)

    def _fill(i, _):
        ones_v[i, :] = jnp.full((_CL,), 1.0, jnp.float32)
        return 0

    def _zero(i, _):
        cnt_v[i, :] = jnp.zeros((_CL,), jnp.float32)
        return 0

    lax.fori_loop(0, _RPC, _fill, 0)
    lax.fori_loop(0, _KPW, _zero, 0)
    pltpu.sync_copy(cnt_v, shared.at[pl.ds(s * _KPW, _KPW)])
    for j in range(_CH):
        pltpu.async_copy(cb_hbm.at[idx_v.at[j]],
                         rows_v.at[pl.ds(j * _RPC, _RPC)], sem).wait()
    pltpu.sync_copy(rows_v, zq_hbm.at[pl.ds(wid * _RPW, _RPW)])
    plsc.subcore_barrier()
    for j in range(_CH):
        pltpu.sync_copy(ones_v, shared.at[idx_v.at[j]], add=True)
    plsc.subcore_barrier()
    pltpu.sync_copy(shared.at[pl.ds(s * _KPW, _KPW)], cnt_v)
    pltpu.sync_copy(cnt_v, cnt_hbm.at[c, pl.ds(s * _KPW, _KPW)])


def _sc_gather_count(codebook, idx3):
    run = pl.kernel(
        _sc_body,
        out_type=[jax.ShapeDtypeStruct((N_ROWS, D), jnp.float32),
                  jax.ShapeDtypeStruct((_NC, K, _CL), jnp.float32)],
        mesh=plsc.VectorSubcoreMesh(core_axis_name="c", subcore_axis_name="s"),
        scratch_types=[pltpu.VMEM((_CH, _RPC), jnp.int32),
                       pltpu.VMEM((_RPW, D), jnp.float32),
                       pltpu.VMEM((_RPC, _CL), jnp.float32),
                       pltpu.VMEM((_KPW, _CL), jnp.float32),
                       pltpu.VMEM_SHARED((K, _CL), jnp.float32),
                       pltpu.SemaphoreType.DMA],
        compiler_params=pltpu.CompilerParams(use_tc_tiling_on_sc=False),
    )
    return run(codebook, idx3)


def _scalars_body(x_ref, zq_ref, cnt_ref, zq_out_ref, loss_ref, perp_ref,
                  util_ref):
    x = x_ref[...]
    zq = zq_ref[...]
    zq_out_ref[...] = jnp.transpose(zq.reshape(16, N_ROWS // 16, D), (0, 2, 1))
    d2 = (x - zq) ** 2
    loss_ref[...] = (jnp.sum(d2) / (N_ROWS * D))[None, None]
    cnt = cnt_ref[0:K // 8, :] + cnt_ref[K // 8:, :]
    p = cnt / N_ROWS
    ent = jnp.sum(p * jnp.log(p + 1e-10)) / _CL
    perp_ref[...] = jnp.exp(-ent)[None, None]
    util_ref[...] = (jnp.sum((cnt > 0).astype(jnp.float32)) / (_CL * K))[None, None]


def _scalars(x_flat, zq_flat, cnt2):
    return pl.pallas_call(
        _scalars_body,
        out_shape=[jax.ShapeDtypeStruct((16, D, N_ROWS // 16), jnp.float32),
                   jax.ShapeDtypeStruct((1, 1), jnp.float32),
                   jax.ShapeDtypeStruct((1, 1), jnp.float32),
                   jax.ShapeDtypeStruct((1, 1), jnp.float32)],
    )(x_flat, zq_flat, cnt2)


def kernel(x, codebook):
    shape = x.shape
    x_flat = x.reshape(-1, D)
    idx2d = _nearest_indices(x_flat, codebook.T)
    idx3 = idx2d.reshape(_NW, _CH, _RPC)
    zq_flat, counts = _sc_gather_count(codebook, idx3)
    cnt2 = counts.reshape(_NC * K * _CL // 128, 128)
    zq_t, loss, perp, util = _scalars(x_flat, zq_flat, cnt2)
    zq_out = jnp.transpose(zq_t, (0, 2, 1))
    del shape
    return (zq_out, idx2d.reshape(x.shape[:-1]),
            loss.reshape(()), perp.reshape(()), util.reshape(()))


# speedup vs baseline: 1.8916x; 1.0007x over previous
import jax
import jax.numpy as jnp
from jax import lax
from jax.experimental import pallas as pl
from jax.experimental.pallas import tpu as pltpu
from jax.experimental.pallas import tpu_sc as plsc

N_ROWS = 9216
D = 64
K = 8192

BM = 1024
GM = N_ROWS // BM
_NBK = 2048
_SB = 128


def _argmax_body(x_ref, cbt_ref, idx_ref, cbn_ref):
    m = pl.program_id(0)

    @pl.when(m == 0)
    def _():
        cbt = cbt_ref[...]
        cbn_ref[...] = cbt / jnp.maximum(
            jnp.sqrt(jnp.sum(cbt * cbt, axis=0, keepdims=True)), 1e-12)

    x = x_ref[...]
    xn = x / jnp.maximum(jnp.sqrt(jnp.sum(x * x, axis=1, keepdims=True)), 1e-12)
    best_v = jnp.full((BM, _SB), -jnp.inf, jnp.float32)
    best_b = jnp.zeros((BM, _SB), jnp.float32)
    for c in range(K // _NBK):
        cbn_c = cbn_ref[:, pl.ds(c * _NBK, _NBK)]
        s = lax.dot_general(xn, cbn_c, (((1,), (0,)), ((), ())),
                            preferred_element_type=jnp.float32,
                            precision=lax.Precision.DEFAULT)
        for j in range(_NBK // _SB):
            blk = s[:, j * _SB:(j + 1) * _SB]
            b_id = jnp.float32(c * (_NBK // _SB) + j)
            gt = blk > best_v
            best_v = jnp.maximum(blk, best_v)
            best_b = jnp.where(gt, b_id, best_b)
    tmax = jnp.max(best_v, axis=1, keepdims=True)
    lane = lax.broadcasted_iota(jnp.int32, (BM, _SB), 1).astype(jnp.float32)
    colf = best_b * _SB + lane
    first = jnp.min(jnp.where(best_v == tmax, colf, jnp.float32(K)),
                    axis=1, keepdims=True)
    idx_ref[...] = first.astype(jnp.int32).reshape(BM // 128, 128)


def _nearest_indices(x_flat, codebook_t):
    return pl.pallas_call(
        _argmax_body,
        grid=(GM,),
        in_specs=[pl.BlockSpec((BM, D), lambda m: (m, 0)),
                  pl.BlockSpec((D, K), lambda m: (0, 0))],
        out_specs=pl.BlockSpec((BM // 128, 128), lambda m: (m, 0)),
        out_shape=jax.ShapeDtypeStruct((N_ROWS // 128, 128), jnp.int32),
        scratch_shapes=[pltpu.VMEM((D, K), jnp.float32)],
    )(x_flat, codebook_t)


_NC = 2
_NS = 16
_NW = _NC * _NS
_RPW = N_ROWS // _NW
_CH = 3
_RPC = _RPW // _CH
_KPW = K // _NS
_CL = 16

def _sc_body(cb_hbm, idx_hbm, zq_hbm, cnt_hbm,
             idx_v, rows_v, ones_v, cnt_v, shared, sem):
    c = lax.axis_index("c")
    s = lax.axis_index("s")
    wid = s * _NC + c
    pltpu.sync_copy(idx_hbm.at[wid], idx_v)

    def _fill(i, _):
        ones_v[i, :] = jnp.full((_CL,), 1.0, jnp.float32)
        return 0

    def _zero(i, _):
        cnt_v[i, :] = jnp.zeros((_CL,), jnp.float32)
        return 0

    lax.fori_loop(0, _RPC, _fill, 0)
    lax.fori_loop(0, _KPW, _zero, 0)
    pltpu.sync_copy(cnt_v, shared.at[pl.ds(s * _KPW, _KPW)])
    for j in range(_CH):
        pltpu.async_copy(cb_hbm.at[idx_v.at[j]],
                         rows_v.at[pl.ds(j * _RPC, _RPC)], sem).wait()
    pltpu.sync_copy(rows_v, zq_hbm.at[pl.ds(wid * _RPW, _RPW)])
    plsc.subcore_barrier()
    for j in range(_CH):
        pltpu.sync_copy(ones_v, shared.at[idx_v.at[j]], add=True)
    plsc.subcore_barrier()
    pltpu.sync_copy(shared.at[pl.ds(s * _KPW, _KPW)], cnt_v)
    pltpu.sync_copy(cnt_v, cnt_hbm.at[c, pl.ds(s * _KPW, _KPW)])


def _sc_gather_count(codebook, idx3):
    run = pl.kernel(
        _sc_body,
        out_type=[jax.ShapeDtypeStruct((N_ROWS, D), jnp.float32),
                  jax.ShapeDtypeStruct((_NC, K, _CL), jnp.float32)],
        mesh=plsc.VectorSubcoreMesh(core_axis_name="c", subcore_axis_name="s"),
        scratch_types=[pltpu.VMEM((_CH, _RPC), jnp.int32),
                       pltpu.VMEM((_RPW, D), jnp.float32),
                       pltpu.VMEM((_RPC, _CL), jnp.float32),
                       pltpu.VMEM((_KPW, _CL), jnp.float32),
                       pltpu.VMEM_SHARED((K, _CL), jnp.float32),
                       pltpu.SemaphoreType.DMA],
        compiler_params=pltpu.CompilerParams(use_tc_tiling_on_sc=False),
    )
    return run(codebook, idx3)


def _scalars_body(x_ref, zq_ref, cnt_ref, zq_out_ref, loss_ref, perp_ref,
                  util_ref):
    x = x_ref[...]
    zq = zq_ref[...]
    zq_out_ref[...] = jnp.transpose(zq.reshape(16, N_ROWS // 16, D), (0, 2, 1))
    d2 = (x - zq) ** 2
    loss_ref[...] = (jnp.sum(d2) / (N_ROWS * D))[None, None]
    cnt = cnt_ref[0:K // 8, :] + cnt_ref[K // 8:, :]
    p = cnt / N_ROWS
    ent = jnp.sum(p * jnp.log(p + 1e-10)) / _CL
    perp_ref[...] = jnp.exp(-ent)[None, None]
    util_ref[...] = (jnp.sum((cnt > 0).astype(jnp.float32)) / (_CL * K))[None, None]


def _scalars(x_flat, zq_flat, cnt2):
    return pl.pallas_call(
        _scalars_body,
        out_shape=[jax.ShapeDtypeStruct((16, D, N_ROWS // 16), jnp.float32),
                   jax.ShapeDtypeStruct((1, 1), jnp.float32),
                   jax.ShapeDtypeStruct((1, 1), jnp.float32),
                   jax.ShapeDtypeStruct((1, 1), jnp.float32)],
    )(x_flat, zq_flat, cnt2)


def kernel(x, codebook):
    shape = x.shape
    x_flat = x.reshape(-1, D)
    idx2d = _nearest_indices(x_flat, codebook.T)
    idx3 = idx2d.reshape(_NW, _CH, _RPC)
    zq_flat, counts = _sc_gather_count(codebook, idx3)
    cnt2 = counts.reshape(_NC * K * _CL // 128, 128)
    zq_t, loss, perp, util = _scalars(x_flat, zq_flat, cnt2)
    zq_out = jnp.transpose(zq_t, (0, 2, 1))
    del shape
    return (zq_out, idx2d.reshape(x.shape[:-1]),
            loss.reshape(()), perp.reshape(()), util.reshape(()))
